# baseline (speedup 1.0000x reference)
import functools

import jax
import jax.numpy as jnp
from jax import lax
from jax.experimental import pallas as pl
from jax.experimental.pallas import tpu as pltpu
from jax.experimental.pallas import tpu_sc as plsc

_N_ITEMS = 50000
_N_USERS = 10000
_EMB = 100
_D = 128
_NI_PAD = 51200
_NU_PAD = 10240
_NC = 2
_NS = 16
_L = 16
_BLK = 2000
_CAP = 128
_CP = pltpu.CompilerParams(needs_layout_passes=False)


def _make_spmm(nnz, n_out_pad, chunk, chunks_per_sc):
    span = nnz // _NS
    nblk = span // _BLK
    ngrp = _BLK // _L
    rows_per_tile = chunk // _NS
    assert span % _BLK == 0 and _BLK % _L == 0 and rows_per_tile % 16 == 0
    mesh = plsc.VectorSubcoreMesh(core_axis_name="c", subcore_axis_name="s")

    @functools.partial(
        pl.kernel,
        out_type=jax.ShapeDtypeStruct((n_out_pad, _D), jnp.float32),
        mesh=mesh,
        compiler_params=_CP,
        scratch_types=[
            pltpu.VMEM_SHARED((chunk, _D), jnp.float32),
            pltpu.VMEM((_BLK,), jnp.int32),
            pltpu.VMEM((_BLK,), jnp.int32),
            pltpu.VMEM((_BLK,), jnp.float32),
            pltpu.VMEM((_CAP + _L,), jnp.int32),
            pltpu.VMEM((_CAP + _L,), jnp.int32),
            pltpu.VMEM((_CAP + _L,), jnp.float32),
            pltpu.VMEM((_CAP,), jnp.int32),
            pltpu.VMEM((_CAP,), jnp.int32),
            pltpu.VMEM((_CAP,), jnp.float32),
            pltpu.VMEM((_CAP, _D), jnp.float32),
            pltpu.SMEM((8,), jnp.int32),
            pltpu.SemaphoreType.DMA,
        ],
    )
    def spmm(row_h, col_h, val_h, x_h, out_h,
             accum, rowb, colb, valb, colf, ridxf, valf, colc, ridxc, valc,
             G, cnt_ref, sem):
        cid = lax.axis_index("c")
        sid = lax.axis_index("s")
        z16i = jnp.zeros((_L,), jnp.int32)
        z16f = jnp.zeros((_L,), jnp.float32)

        for i in range((_CAP + _L) // _L):
            colf[pl.ds(i * _L, _L)] = z16i
            ridxf[pl.ds(i * _L, _L)] = z16i
            valf[pl.ds(i * _L, _L)] = z16f
        cnt_ref[0] = 0
        cnt_ref[1] = 0

        def issue():
            for i in range(_CAP // _L):
                colc[pl.ds(i * _L, _L)] = colf[pl.ds(i * _L, _L)]
                ridxc[pl.ds(i * _L, _L)] = ridxf[pl.ds(i * _L, _L)]
                valc[pl.ds(i * _L, _L)] = valf[pl.ds(i * _L, _L)]
            pltpu.async_copy(x_h.at[colc], G, sem)
            for i in range(_CAP // _L):
                valf[pl.ds(i * _L, _L)] = z16f
            cnt_ref[0] = 0
            cnt_ref[1] = 1

        def complete():
            pltpu.make_async_copy(x_h.at[colc], G, sem).wait()

            def _scale(g, c):
                vv = valc[pl.ds(g * _L, _L)]
                for r in range(_L):
                    j = g * _L + r
                    vs = z16f + vv[r]
                    for d in range(_D // _L):
                        G[j, pl.ds(d * _L, _L)] = G[j, pl.ds(d * _L, _L)] * vs
                return c
            lax.fori_loop(0, _CAP // _L, _scale, 0)

            pltpu.sync_copy(G, accum.at[ridxc], add=True)
            cnt_ref[1] = 0

        zfull, zrem = divmod(rows_per_tile, _CAP)
        for ci in range(chunks_per_sc):
            base = (cid * chunks_per_sc + ci) * chunk
            row0 = sid * rows_per_tile
            def _zg(i, c):
                for d in range(_D // _L):
                    G[i, pl.ds(d * _L, _L)] = z16f
                return c
            lax.fori_loop(0, _CAP, _zg, 0)
            for z in range(zfull):
                pltpu.sync_copy(G, accum.at[pl.ds(row0 + z * _CAP, _CAP)])
            if zrem:
                pltpu.sync_copy(G.at[pl.ds(0, zrem)],
                                accum.at[pl.ds(row0 + zfull * _CAP, zrem)])
            plsc.subcore_barrier()

            tile_lo = sid * span

            def _grp(g, c):
                rv = rowb[pl.ds(g * _L, _L)]
                m = (rv >= base) & (rv < base + chunk)

                @pl.when(cnt_ref[0] > _CAP - _L)
                def _():
                    @pl.when(cnt_ref[1] == 1)
                    def _():
                        complete()
                    issue()

                cnt = cnt_ref[0]
                cs = lax.cumsum(m.astype(jnp.int32))
                pos = jnp.where(m, cnt + cs - 1, _CAP)
                plsc.store_scatter(colf, [pos], colb[pl.ds(g * _L, _L)])
                plsc.store_scatter(ridxf, [pos], rv - base)
                plsc.store_scatter(valf, [pos], valb[pl.ds(g * _L, _L)])
                cnt_ref[0] = cnt + cs[_L - 1]
                return c

            def _blk(b, c):
                off = tile_lo + b * _BLK
                pltpu.sync_copy(row_h.at[pl.ds(off, _BLK)], rowb)
                pltpu.sync_copy(col_h.at[pl.ds(off, _BLK)], colb)
                pltpu.sync_copy(val_h.at[pl.ds(off, _BLK)], valb)
                lax.fori_loop(0, ngrp, _grp, 0)
                return c
            lax.fori_loop(0, nblk, _blk, 0)

            @pl.when(cnt_ref[1] == 1)
            def _():
                complete()

            @pl.when(cnt_ref[0] > 0)
            def _():
                issue()
                complete()
            plsc.subcore_barrier()

            for z in range(zfull):
                pltpu.sync_copy(accum.at[pl.ds(row0 + z * _CAP, _CAP)],
                                out_h.at[pl.ds(base + row0 + z * _CAP, _CAP)])
            if zrem:
                pltpu.sync_copy(
                    accum.at[pl.ds(row0 + zfull * _CAP, zrem)],
                    out_h.at[pl.ds(base + row0 + zfull * _CAP, zrem)])

    return spmm


_spmm_adj = _make_spmm(800000, _NI_PAD, 12800, 2)
_spmm_usr = _make_spmm(320000, _NU_PAD, 5120, 1)

_gmesh = plsc.VectorSubcoreMesh(core_axis_name="c", subcore_axis_name="s")


@functools.partial(
    pl.kernel,
    out_type=jax.ShapeDtypeStruct((1024, _D), jnp.float32),
    mesh=_gmesh,
    compiler_params=_CP,
    scratch_types=[
        pltpu.VMEM((32,), jnp.int32),
        pltpu.VMEM((32, _D), jnp.float32),
        pltpu.SemaphoreType.DMA,
    ],
)
def _gather_users(user_h, tab_h, out_h, idxb, g32, sem):
    wid = lax.axis_index("s") * _NC + lax.axis_index("c")
    b0 = wid * 32
    pltpu.sync_copy(user_h.at[pl.ds(b0, 32)], idxb)
    pltpu.async_copy(tab_h.at[idxb], g32, sem).wait()
    pltpu.sync_copy(g32, out_h.at[pl.ds(b0, 32)])


def _sum4(a, b, c, d):
    def body(a_r, b_r, c_r, d_r, o_r):
        o_r[...] = a_r[...] + b_r[...] + c_r[...] + d_r[...]
    n = a.shape[0]
    blkr = 512
    return pl.pallas_call(
        body,
        grid=(n // blkr,),
        in_specs=[pl.BlockSpec((blkr, 128), lambda i: (i, 0))] * 4,
        out_specs=pl.BlockSpec((blkr, 128), lambda i: (i, 0)),
        out_shape=jax.ShapeDtypeStruct((n, 128), jnp.float32),
    )(a, b, c, d)


def kernel(adj_row, adj_col, adj_val, u_row, u_col, u_val, ishist,
           hist_item, hist_len, embedding, user_embedding, user):
    adj_row = adj_row.astype(jnp.int32)
    adj_col = adj_col.astype(jnp.int32)
    u_row = u_row.astype(jnp.int32)
    u_col = u_col.astype(jnp.int32)
    user = user.astype(jnp.int32)

    x0 = jnp.pad(embedding, ((0, _NI_PAD - _N_ITEMS), (0, _D - _EMB)))
    c1 = _spmm_adj(adj_row, adj_col, adj_val, x0)
    c2 = _spmm_adj(adj_row, adj_col, adj_val, c1)
    c3 = _spmm_adj(adj_row, adj_col, adj_val, c2)
    fin = _sum4(x0, c1, c2, c3)
    utab = _spmm_usr(u_row, u_col, u_val, fin)
    ue = _gather_users(user, utab)
    return fin[:_N_ITEMS, :_EMB], ue[:, :_EMB]
